# baseline (speedup 1.0000x reference)
import functools

import jax
import jax.numpy as jnp
from jax import lax
from jax.experimental import pallas as pl
from jax.experimental.pallas import tpu as pltpu
from jax.experimental.pallas import tpu_sc as plsc


def _zero_fill(B, W):
    def body(o_ref):
        o_ref[...] = jnp.zeros_like(o_ref)

    blk = 64
    return pl.pallas_call(
        body,
        out_shape=jax.ShapeDtypeStruct((B, W), jnp.float32),
        grid=(B // blk,),
        out_specs=pl.BlockSpec((blk, W), lambda i: (i, 0)),
    )()


def _build_scatter(B, D, H):
    NC, NS, L = 2, 16, 16
    NW = NC * NS
    N = B * H
    E = N // NW
    G = E // 128
    assert E % 128 == 0 and N % NW == 0

    mesh = plsc.VectorSubcoreMesh(
        core_axis_name="c", subcore_axis_name="s", num_cores=NC, num_subcores=NS
    )

    @functools.partial(
        pl.kernel,
        mesh=mesh,
        out_type=(),
        scratch_types=[
            pltpu.VMEM((E,), jnp.int32),
            pltpu.VMEM((G, 128), jnp.int32),
            pltpu.VMEM((128,), jnp.float32),
            pltpu.SemaphoreType.DMA,
        ],
    )
    def scatter(x_hbm, out_ref, xv, addrv, onesv, sem):
        wid = lax.axis_index("s") * NC + lax.axis_index("c")
        base = wid * E
        pltpu.sync_copy(x_hbm.at[pl.ds(base, E)], xv)

        for t in range(128 // L):
            onesv[pl.ds(t * L, L)] = jnp.full((L,), 1.0, jnp.float32)

        def fill_group(g, carry):
            for kk in range(128 // L):
                off = g * 128 + kk * L
                i = base + off + lax.iota(jnp.int32, L)
                x = xv[pl.ds(off, L)]
                b = ((i.astype(jnp.float32) + 0.5) * (1.0 / H)).astype(
                    jnp.int32
                )
                h = i - b * H
                addrv[g, pl.ds(kk * L, L)] = b * (D * H) + x * H + h
            return carry

        lax.fori_loop(0, G, fill_group, 0)

        descs = [
            pltpu.async_copy(onesv, out_ref.at[addrv.at[j]], sem)
            for j in range(G)
        ]
        for d in descs:
            d.wait()

    return scatter


def kernel(X_in, ones):
    D = ones.shape[0]
    B, H = X_in.shape
    zeros = _zero_fill(B, D * H).reshape(B * D * H)
    out_ref = jax.new_ref(zeros)
    _build_scatter(B, D, H)(X_in.reshape(B * H), out_ref)
    return out_ref[...].reshape(B, D, H)


# speedup vs baseline: 1.0009x; 1.0009x over previous
import jax
import jax.numpy as jnp
from jax import lax
from jax.experimental import pallas as pl
from jax.experimental.pallas import tpu as pltpu
from jax.experimental.pallas import tpu_sc as plsc
from jax._src.pallas import mpmd as _pl_mpmd


def _zero_fill(B, W):
    def body(o_ref):
        o_ref[...] = jnp.zeros_like(o_ref)

    blk = 64
    return pl.pallas_call(
        body,
        out_shape=jax.ShapeDtypeStruct((B, W), jnp.float32),
        grid=(B // blk,),
        out_specs=pl.BlockSpec((blk, W), lambda i: (i, 0)),
    )()


def _build_scatter(B, D, H):
    NC, NS, L = 2, 16, 16
    NW = NC * NS
    N = B * H
    E = N // NW
    G = E // 128
    assert E % 128 == 0 and N % NW == 0

    mesh = plsc.VectorSubcoreMesh(
        core_axis_name="c", subcore_axis_name="s", num_cores=NC, num_subcores=NS
    )

    def scatter(z_hbm, x_hbm, out_ref, xv, addrv, onesv, sem):
        del z_hbm
        wid = lax.axis_index("s") * NC + lax.axis_index("c")
        base = wid * E
        pltpu.sync_copy(x_hbm.at[pl.ds(base, E)], xv)

        for t in range(128 // L):
            onesv[pl.ds(t * L, L)] = jnp.full((L,), 1.0, jnp.float32)

        def fill_group(g, carry):
            for kk in range(128 // L):
                off = g * 128 + kk * L
                i = base + off + lax.iota(jnp.int32, L)
                x = xv[pl.ds(off, L)]
                b = ((i.astype(jnp.float32) + 0.5) * (1.0 / H)).astype(
                    jnp.int32
                )
                h = i - b * H
                addrv[g, pl.ds(kk * L, L)] = b * (D * H) + x * H + h
            return carry

        lax.fori_loop(0, G, fill_group, 0)

        descs = [
            pltpu.async_copy(onesv, out_ref.at[addrv.at[j]], sem)
            for j in range(G)
        ]
        for d in descs:
            d.wait()

    return _pl_mpmd._mpmd_map(
        [(mesh, scatter)],
        jax.ShapeDtypeStruct((B * D * H,), jnp.float32),
        input_output_aliases={0: 0},
        scratch_types=[
            pltpu.VMEM((E,), jnp.int32),
            pltpu.VMEM((G, 128), jnp.int32),
            pltpu.VMEM((128,), jnp.float32),
            pltpu.SemaphoreType.DMA,
        ],
    )


def kernel(X_in, ones):
    D = ones.shape[0]
    B, H = X_in.shape
    zeros = _zero_fill(B, D * H).reshape(B * D * H)
    out = _build_scatter(B, D, H)(zeros, X_in.reshape(B * H))
    return out.reshape(B, D, H)


# speedup vs baseline: 59.6821x; 59.6280x over previous
import jax
import jax.numpy as jnp
from jax import lax
from jax.experimental import pallas as pl


def _build_one_hot_t(B, D, H, BB):
    def body(x_ref, o_ref):
        i = pl.program_id(0)
        x = x_ref[:, pl.ds(i * BB, BB)]
        d = lax.broadcasted_iota(jnp.int32, (1, D, 1), 1)
        o_ref[...] = (x[:, None, :] == d).astype(jnp.float32)

    return pl.pallas_call(
        body,
        out_shape=jax.ShapeDtypeStruct((H, D, B), jnp.float32),
        grid=(B // BB,),
        in_specs=[pl.BlockSpec((H, B), lambda i: (0, 0))],
        out_specs=pl.BlockSpec((H, D, BB), lambda i: (0, 0, i)),
    )


def kernel(X_in, ones):
    D = ones.shape[0]
    B, H = X_in.shape
    out_t = _build_one_hot_t(B, D, H, BB=128)(X_in.T)
    return jnp.transpose(out_t, (2, 1, 0))
